# Initial kernel scaffold; baseline (speedup 1.0000x reference)
import jax
import jax.numpy as jnp
from jax.experimental import pallas as pl


def kernel(team_idx, weight):
    raise NotImplementedError("write your pallas kernel here")


# speedup vs baseline: 3.0000x; 3.0000x over previous
import functools

import jax
import jax.numpy as jnp
from jax import lax
from jax.experimental import pallas as pl
from jax.experimental.pallas import tpu as pltpu
from jax.experimental.pallas import tpu_sc as plsc

_BATCH = 16384
_FIELDS = 26
_D = 128
_NUM_ROWS = _BATCH * _FIELDS
_NC = 2
_NS = 16
_NW = _NC * _NS
_B_PER_W = _NUM_ROWS // _NW
_CHUNK = 128
_N_CHUNKS = _B_PER_W // _CHUNK


@jax.jit
def _sc_gather(idx, weight):
    mesh = plsc.VectorSubcoreMesh(core_axis_name="c", subcore_axis_name="s")

    @functools.partial(
        pl.kernel,
        out_type=jax.ShapeDtypeStruct((_NUM_ROWS, _D), jnp.float32),
        mesh=mesh,
        scratch_types=[
            pltpu.VMEM((_N_CHUNKS, _CHUNK), jnp.int32),
            pltpu.VMEM((_CHUNK, _D), jnp.float32),
            pltpu.SemaphoreType.DMA,
        ],
    )
    def k(idx_hbm, w_hbm, out_hbm, idx_v, rows_v, sem):
        wid = lax.axis_index("s") * _NC + lax.axis_index("c")
        base = wid * _B_PER_W
        pltpu.sync_copy(idx_hbm.at[wid], idx_v)

        def body(j, carry):
            pltpu.async_copy(w_hbm.at[idx_v.at[j]], rows_v, sem).wait()
            pltpu.sync_copy(rows_v, out_hbm.at[pl.ds(base + j * _CHUNK, _CHUNK)])
            return carry

        lax.fori_loop(0, _N_CHUNKS, body, 0)

    return k(idx, weight)


def kernel(team_idx, weight):
    idx = team_idx.astype(jnp.int32).reshape(_NW, _N_CHUNKS, _CHUNK)
    out = _sc_gather(idx, weight)
    return out.reshape(_BATCH, _FIELDS, _D)


# speedup vs baseline: 3.3865x; 1.1288x over previous
import functools

import jax
import jax.numpy as jnp
from jax import lax
from jax.experimental import pallas as pl
from jax.experimental.pallas import tpu as pltpu
from jax.experimental.pallas import tpu_sc as plsc

_BATCH = 16384
_FIELDS = 26
_D = 128
_NUM_ROWS = _BATCH * _FIELDS
_NC = 2
_NS = 16
_NW = _NC * _NS
_B_PER_W = _NUM_ROWS // _NW
_CHUNK = 128
_N_CHUNKS = _B_PER_W // _CHUNK
_NBUF = 4
_N_GROUPS = _N_CHUNKS // _NBUF


@jax.jit
def _sc_gather(idx, weight):
    mesh = plsc.VectorSubcoreMesh(core_axis_name="c", subcore_axis_name="s")

    @functools.partial(
        pl.kernel,
        out_type=jax.ShapeDtypeStruct((_NUM_ROWS, _D), jnp.float32),
        mesh=mesh,
        scratch_types=[
            pltpu.VMEM((_N_CHUNKS, _CHUNK), jnp.int32),
            pltpu.VMEM((_NBUF, _CHUNK, _D), jnp.float32),
            pltpu.SemaphoreType.DMA,
            pltpu.SemaphoreType.DMA,
        ],
    )
    def k(idx_hbm, w_hbm, out_hbm, idx_v, bufs, sem_g, sem_s):
        wid = lax.axis_index("s") * _NC + lax.axis_index("c")
        base = wid * _B_PER_W
        pltpu.sync_copy(idx_hbm.at[wid], idx_v)

        def group(g, drain_scatters):
            if drain_scatters:
                for b in range(_NBUF):
                    pltpu.make_async_copy(
                        bufs.at[b],
                        out_hbm.at[pl.ds(base, _CHUNK)], sem_s).wait()
            gd = []
            for b in range(_NBUF):
                gd.append(pltpu.async_copy(
                    w_hbm.at[idx_v.at[g * _NBUF + b]], bufs.at[b], sem_g))
            for b in range(_NBUF):
                gd[b].wait()
                pltpu.async_copy(
                    bufs.at[b],
                    out_hbm.at[pl.ds(base + (g * _NBUF + b) * _CHUNK, _CHUNK)],
                    sem_s)

        group(0, drain_scatters=False)

        def body(t, carry):
            group(t, drain_scatters=True)
            return carry

        lax.fori_loop(1, _N_GROUPS, body, 0)

        for b in range(_NBUF):
            pltpu.make_async_copy(
                bufs.at[b], out_hbm.at[pl.ds(base, _CHUNK)], sem_s).wait()

    return k(idx, weight)


def kernel(team_idx, weight):
    idx = team_idx.astype(jnp.int32).reshape(_NW, _N_CHUNKS, _CHUNK)
    out = _sc_gather(idx, weight)
    return out.reshape(_BATCH, _FIELDS, _D)


# speedup vs baseline: 11.8381x; 3.4957x over previous
import functools

import jax
import jax.numpy as jnp
from jax import lax
from jax.experimental import pallas as pl
from jax.experimental.pallas import tpu as pltpu
from jax.experimental.pallas import tpu_sc as plsc

_BATCH = 16384
_FIELDS = 26
_D = 128
_NUM_ROWS = _BATCH * _FIELDS
_NC = 2
_NS = 16
_NW = _NC * _NS
_B_PER_W = _NUM_ROWS // _NW
_CHUNK = 128
_N_CHUNKS = _B_PER_W // _CHUNK
_SUPER = 2 * _CHUNK
_N_SUPER = _B_PER_W // _SUPER
_NBUF = 3
_N_GROUPS = _N_SUPER // _NBUF
_N_TAIL = _N_SUPER - _N_GROUPS * _NBUF


@jax.jit
def _sc_gather(idx, weight):
    mesh = plsc.VectorSubcoreMesh(core_axis_name="c", subcore_axis_name="s")

    @functools.partial(
        pl.kernel,
        out_type=jax.ShapeDtypeStruct((_NUM_ROWS, _D), jnp.float32),
        mesh=mesh,
        scratch_types=[
            pltpu.VMEM((_N_CHUNKS, _CHUNK), jnp.int32),
            pltpu.VMEM((_NBUF, _SUPER, _D), jnp.float32),
            pltpu.SemaphoreType.DMA,
            pltpu.SemaphoreType.DMA,
        ],
    )
    def k(idx_hbm, w_hbm, out_hbm, idx_v, bufs, sem_g, sem_s):
        wid = lax.axis_index("s") * _NC + lax.axis_index("c")
        base = wid * _B_PER_W
        pltpu.sync_copy(idx_hbm.at[wid], idx_v)

        def drain_one_scatter():
            pltpu.make_async_copy(
                bufs.at[0], out_hbm.at[pl.ds(base, _SUPER)], sem_s).wait()

        def group(first, nsuper, ndrain):
            gd = []
            for b in range(nsuper):
                if b < ndrain:
                    drain_one_scatter()
                s = first + b
                gd.append((
                    pltpu.async_copy(w_hbm.at[idx_v.at[2 * s]],
                                     bufs.at[b, pl.ds(0, _CHUNK)], sem_g),
                    pltpu.async_copy(w_hbm.at[idx_v.at[2 * s + 1]],
                                     bufs.at[b, pl.ds(_CHUNK, _CHUNK)], sem_g),
                ))
            for b in range(nsuper):
                gd[b][0].wait()
                gd[b][1].wait()
                pltpu.async_copy(
                    bufs.at[b],
                    out_hbm.at[pl.ds(base + (first + b) * _SUPER, _SUPER)],
                    sem_s)

        group(0, _NBUF, 0)

        def body(t, carry):
            group(t * _NBUF, _NBUF, _NBUF)
            return carry

        lax.fori_loop(1, _N_GROUPS, body, 0)

        group(_N_GROUPS * _NBUF, _N_TAIL, _N_TAIL)
        for _ in range(_NBUF):
            drain_one_scatter()

    return k(idx, weight)


def kernel(team_idx, weight):
    idx = team_idx.astype(jnp.int32).T.reshape(_NW, _N_CHUNKS, _CHUNK)
    out = _sc_gather(idx, weight)
    return out.reshape(_FIELDS, _BATCH, _D).transpose(1, 0, 2)
